# baseline (speedup 1.0000x reference)
import jax
import jax.numpy as jnp
from jax import lax
from jax.experimental import pallas as pl
from jax.experimental.pallas import tpu as pltpu
from jax.experimental.pallas import tpu_sc as plsc

N = 100000
DV = 16
CN = 50000
DC = 32
NT = 16

VPT = 6256
NP = NT * VPT
EPT = VPT * DV
EP = NP * DV
GPT = 196
G = NT * GPT
CNP = G * 16
IW = 512
ROWS_PT = GPT
N_ROUNDS = 4
RPC = 2
NCH_A = GPT // RPC
EBW = 17 * 256
NCH_B = VPT // 16 // 17
SBIT = -2147483648
EPS = 1e-12


def _body(cnp_h, llr16_h, llr0p_h, gam_h, fidx_h, out_h,
          idxba, msg2a, vals2a,
          ebuf, lbuf, obuf, c2v_v, gam_v, V_s, sem):
    cid = lax.axis_index("c")
    sid = lax.axis_index("s")

    @pl.when(cid == 0)
    def _work():
        t = sid
        eoff = t * EPT

        pltpu.sync_copy(gam_h, gam_v)
        pltpu.sync_copy(llr16_h.at[pl.ds(eoff, EPT)], V_s.at[pl.ds(eoff, EPT)])
        plsc.subcore_barrier()

        iota16 = lax.iota(jnp.int32, 16)
        gamv = gam_v[...]
        sbit = jnp.full((16,), SBIT, jnp.int32)

        def idx_load(h, c, dst):
            pltpu.sync_copy(
                h.at[pl.ds((t * ROWS_PT + c * RPC) * IW, RPC * IW)], dst)

        def fire_gather(ib, mb, s):
            return [pltpu.async_copy(V_s.at[ib.at[pl.ds(r * IW, IW)]],
                                     mb.at[pl.ds(r * IW, IW)], s)
                    for r in range(RPC)]

        def fire_scatter(ib, vb, s):
            return [pltpu.async_copy(vb.at[pl.ds(r * IW, IW)],
                                     V_s.at[ib.at[pl.ds(r * IW, IW)]],
                                     s, add=True)
                    for r in range(RPC)]

        def drain(cps):
            for cp in cps:
                cp.wait()

        def reduce_chunk(c, mb):
            for g in range(RPC):
                sacc = None
                mag = None
                for k in range(DC):
                    m = mb[pl.ds(g * IW + k * 16, 16)]
                    sb = lax.bitcast_convert_type(m + EPS, jnp.int32)
                    am = jnp.abs(m)
                    if sacc is None:
                        sacc, mag = sb, am
                    else:
                        sacc = lax.bitwise_xor(sacc, sb)
                        mag = jnp.minimum(mag, am)
                sgn = lax.bitwise_and(sacc, sbit)
                c2v = lax.bitcast_convert_type(
                    lax.bitwise_xor(
                        lax.bitcast_convert_type(gamv * mag, jnp.int32), sgn),
                    jnp.float32)
                cids = t * (GPT * 16) + (c * RPC + g) * 16 + iota16
                c2v = jnp.where(cids < CN, c2v, 0.0)
                c2v_v[pl.ds((c * RPC + g) * 16, 16)] = c2v

        def build_chunk(c, vb):
            for g in range(RPC):
                cv = c2v_v[pl.ds((c * RPC + g) * 16, 16)]
                for k in range(DC):
                    vb[pl.ds(g * IW + k * 16, 16)] = cv

        def _round(r_, rc):
            def _ga(c, cc):
                idx_load(cnp_h, c, idxba)
                drain(fire_gather(idxba, msg2a, sem))
                reduce_chunk(c, msg2a)
                return cc
            lax.fori_loop(0, NCH_A, _ga, 0)
            plsc.subcore_barrier()

            def _pb(i, cc):
                off = eoff + i * EBW
                pltpu.sync_copy(V_s.at[pl.ds(off, EBW)], ebuf)
                pltpu.sync_copy(llr16_h.at[pl.ds(off, EBW)], lbuf)

                def _sub(j, c2):
                    sl = pl.ds(j * 16, 16)
                    ebuf[sl] = lbuf[sl] - ebuf[sl]
                    return c2
                lax.fori_loop(0, EBW // 16, _sub, 0)
                pltpu.sync_copy(ebuf, V_s.at[pl.ds(off, EBW)])
                return cc
            lax.fori_loop(0, NCH_B, _pb, 0)
            plsc.subcore_barrier()

            def _gc(c, cc):
                idx_load(cnp_h, c, idxba)
                build_chunk(c, vals2a)
                drain(fire_scatter(idxba, vals2a, sem))
                return cc
            lax.fori_loop(0, NCH_A, _gc, 0)
            plsc.subcore_barrier()
            return rc
        lax.fori_loop(0, N_ROUNDS, _round, 0)

        def sum_chunk(c, mb):
            for r in range(RPC):
                for h in range(2):
                    acc = mb[pl.ds(r * IW + h * 256, 16)]
                    for j in range(1, DV):
                        acc = acc + mb[pl.ds(r * IW + h * 256 + j * 16, 16)]
                    w = (c * RPC + r) * 2 + h
                    sl = pl.ds(w * 16, 16)
                    obuf[sl] = obuf[sl] + acc

        pltpu.sync_copy(llr0p_h.at[pl.ds(t * VPT, VPT)], obuf.at[pl.ds(0, VPT)])

        def _fin(c, cc):
            idx_load(fidx_h, c, idxba)
            drain(fire_gather(idxba, msg2a, sem))
            sum_chunk(c, msg2a)
            return cc
        lax.fori_loop(0, NCH_A, _fin, 0)
        pltpu.sync_copy(obuf.at[pl.ds(0, VPT)], out_h.at[pl.ds(t * VPT, VPT)])


_bp_call = pl.kernel(
    _body,
    out_type=jax.ShapeDtypeStruct((NP,), jnp.float32),
    mesh=plsc.VectorSubcoreMesh(core_axis_name="c", subcore_axis_name="s"),
    scratch_types=[
        pltpu.VMEM((RPC * IW,), jnp.int32),
        pltpu.VMEM((RPC * IW,), jnp.float32),
        pltpu.VMEM((RPC * IW,), jnp.float32),
        pltpu.VMEM((EBW,), jnp.float32),
        pltpu.VMEM((EBW,), jnp.float32),
        pltpu.VMEM((VPT + 16,), jnp.float32),
        pltpu.VMEM((GPT * 16,), jnp.float32),
        pltpu.VMEM((16,), jnp.float32),
        pltpu.VMEM_SHARED((EP,), jnp.float32),
        pltpu.SemaphoreType.DMA,
    ],
)


def kernel(llr0, vn_adj, cn_adj, gamma):
    del vn_adj
    llr0p = jnp.concatenate([llr0, jnp.zeros((NP - N,), llr0.dtype)])
    llr16 = jnp.broadcast_to(llr0p[:, None], (NP, DV)).reshape(-1)
    cn_pad = jnp.concatenate(
        [cn_adj, jnp.zeros((CNP - CN, DC), cn_adj.dtype)])
    cnp = cn_pad.reshape(G, 16, DC).transpose(0, 2, 1).reshape(-1)
    gamma16 = jnp.full((16,), gamma, jnp.float32)
    fidx = (jnp.arange(NT, dtype=jnp.int32)[:, None, None, None] * EPT
            + jnp.arange(VPT // 16, dtype=jnp.int32)[None, :, None, None] * 256
            + jnp.arange(DV, dtype=jnp.int32)[None, None, :, None]
            + jnp.arange(16, dtype=jnp.int32)[None, None, None, :] * 16)
    fidx = fidx.reshape(NT, -1)
    fidx = jnp.concatenate(
        [fidx, jnp.zeros((NT, ROWS_PT * IW - fidx.shape[1]), jnp.int32)],
        axis=1).reshape(-1)
    out = _bp_call(cnp, llr16, llr0p, gamma16, fidx)
    return out[:N]
